# baseline (speedup 1.0000x reference)
import functools

import jax
import jax.numpy as jnp
from jax import lax
from jax.experimental import pallas as pl
from jax.experimental.pallas import tpu as pltpu
from jax.experimental.pallas import tpu_sc as plsc

_LANES = 128


@functools.lru_cache(maxsize=None)
def _build(n_idx, vocab, dim):
    info = plsc.get_sparse_core_info()
    nc, ns, nl = info.num_cores, info.num_subcores, info.num_lanes
    nw = nc * ns
    rows_total = n_idx // _LANES
    rows_per_w = rows_total // nw
    npair = rows_per_w // 2
    orpc = _LANES * dim // 128

    mesh = plsc.VectorSubcoreMesh(core_axis_name="c", subcore_axis_name="s")

    @functools.partial(
        pl.kernel,
        mesh=mesh,
        out_type=jax.ShapeDtypeStruct((n_idx * dim // 128, 128), jnp.float32),
        scratch_types=[
            pltpu.VMEM((rows_per_w, _LANES), jnp.int32),
            pltpu.VMEM((2, _LANES), jnp.int32),
            pltpu.VMEM((2, _LANES), jnp.int32),
            pltpu.VMEM((2, _LANES, 128), jnp.float32),
            pltpu.VMEM((2, _LANES, nl), jnp.float32),
            pltpu.VMEM((2, orpc, 128), jnp.float32),
            pltpu.SemaphoreType.DMA,
            pltpu.SemaphoreType.DMA,
            pltpu.SemaphoreType.DMA,
            pltpu.SemaphoreType.DMA,
        ],
        compiler_params=pltpu.CompilerParams(use_tc_tiling_on_sc=False),
    )
    def emb(idx_hbm, table2_hbm, const_hbm, out2_hbm, idx_v, q_v, h_v,
            pair_v, hexp_v, emb_v, sem0, sem1, wsem0, wsem1):
        wid = lax.axis_index("s") * nc + lax.axis_index("c")
        row0 = wid * rows_per_w
        sems = (sem0, sem1)
        wsems = (wsem0, wsem1)

        pltpu.sync_copy(idx_hbm.at[pl.ds(row0, rows_per_w)], idx_v)

        def prep(c, b):
            for t in range(_LANES // nl):
                v = idx_v[c, pl.ds(t * nl, nl)]
                q_v[b, pl.ds(t * nl, nl)] = lax.shift_right_logical(v, 1)
                h_v[b, pl.ds(t * nl, nl)] = v & 1

        def fire(c, b):
            pltpu.async_copy(table2_hbm.at[q_v.at[b]], pair_v.at[b], sems[b])
            pltpu.async_copy(const_hbm.at[h_v.at[b]], hexp_v.at[b],
                             sems[b])

        def drain(c, b):
            pltpu.make_async_copy(
                table2_hbm.at[q_v.at[b]], pair_v.at[b], sems[b]).wait()
            pltpu.make_async_copy(
                const_hbm.at[h_v.at[b]], hexp_v.at[b], sems[b]).wait()

        def blend(c, b):
            for j in range(_LANES):
                h = hexp_v[b, j, pl.ds(0, nl)]
                for t in range(dim // nl):
                    lo = pair_v[b, j, pl.ds(t * nl, nl)]
                    hi = pair_v[b, j, pl.ds(dim + t * nl, nl)]
                    emb_v[b, j >> 1, pl.ds(dim * (j & 1) + t * nl, nl)] = (
                        lo + (hi - lo) * h)

        def put(c, b):
            pltpu.async_copy(
                emb_v.at[b],
                out2_hbm.at[pl.ds((row0 + c) * orpc, orpc)], wsems[b])

        def put_wait(c, b):
            pltpu.make_async_copy(
                emb_v.at[b],
                out2_hbm.at[pl.ds((row0 + c) * orpc, orpc)], wsems[b]).wait()

        prep(0, 0)
        fire(0, 0)
        prep(1, 1)
        fire(1, 1)

        def pair_step(p, _):
            c0 = p * 2
            for b in range(2):
                c = c0 + b

                @pl.when(p > 0)
                def _():
                    put_wait(c - 2, b)

                drain(c, b)
                blend(c, b)
                put(c, b)

                @pl.when(p + 1 < npair)
                def _():
                    prep(c + 2, b)
                    fire(c + 2, b)

            return 0

        lax.fori_loop(0, npair, pair_step, 0)
        put_wait(rows_per_w - 2, 0)
        put_wait(rows_per_w - 1, 1)

    return emb


def kernel(inputs, table):
    seq, batch = inputs.shape
    vocab, dim = table.shape
    n_idx = seq * batch
    flat_idx = inputs.reshape(n_idx // _LANES, _LANES)
    table2 = table.reshape(vocab * dim // 128, 128)
    const2 = jnp.concatenate([jnp.zeros((1, 16), jnp.float32),
                              jnp.ones((1, 16), jnp.float32)])
    out = _build(n_idx, vocab, dim)(flat_idx, table2, const2)
    return out.reshape(seq, batch, dim)


# speedup vs baseline: 2.7809x; 2.7809x over previous
import functools

import jax
import jax.numpy as jnp
from jax import lax
from jax.experimental import pallas as pl
from jax.experimental.pallas import tpu as pltpu
from jax.experimental.pallas import tpu_sc as plsc

_LANES = 128


@functools.lru_cache(maxsize=None)
def _build(n_idx, vocab, dim):
    info = plsc.get_sparse_core_info()
    nc, ns, nl = info.num_cores, info.num_subcores, info.num_lanes
    nw = nc * ns
    rows_total = n_idx // _LANES
    rows_per_w = rows_total // nw
    npair = rows_per_w // 2
    orpc = _LANES * dim // 128

    mesh = plsc.VectorSubcoreMesh(core_axis_name="c", subcore_axis_name="s")

    @functools.partial(
        pl.kernel,
        mesh=mesh,
        out_type=jax.ShapeDtypeStruct((n_idx * dim // 128, 128), jnp.float32),
        scratch_types=[
            pltpu.VMEM((rows_per_w, _LANES), jnp.int32),
            pltpu.VMEM((2, _LANES), jnp.int32),
            pltpu.VMEM((2, _LANES), jnp.int32),
            pltpu.VMEM((2, _LANES, 128), jnp.float32),
            pltpu.VMEM((2, _LANES, nl), jnp.float32),
            pltpu.VMEM((2, orpc, 128), jnp.float32),
            pltpu.SemaphoreType.DMA,
            pltpu.SemaphoreType.DMA,
            pltpu.SemaphoreType.DMA,
            pltpu.SemaphoreType.DMA,
        ],
        compiler_params=pltpu.CompilerParams(use_tc_tiling_on_sc=False),
    )
    def emb(idx_hbm, table2_hbm, const_hbm, out2_hbm, idx_v, q_v, h_v,
            pair_v, hexp_v, emb_v, sem0, sem1, wsem0, wsem1):
        wid = lax.axis_index("s") * nc + lax.axis_index("c")
        row0 = wid * rows_per_w
        sems = (sem0, sem1)
        wsems = (wsem0, wsem1)

        pltpu.sync_copy(idx_hbm.at[pl.ds(row0, rows_per_w)], idx_v)

        def prep(c, b):
            for t in range(_LANES // nl):
                v = idx_v[c, pl.ds(t * nl, nl)]
                q_v[b, pl.ds(t * nl, nl)] = lax.shift_right_logical(v, 1)
                h_v[b, pl.ds(t * nl, nl)] = (v & 1) + (2 * wid)

        def fire(c, b):
            pltpu.async_copy(table2_hbm.at[q_v.at[b]], pair_v.at[b], sems[b])
            pltpu.async_copy(const_hbm.at[h_v.at[b]], hexp_v.at[b], sems[b])

        def drain(c, b):
            pltpu.make_async_copy(
                table2_hbm.at[q_v.at[b]], pair_v.at[b], sems[b]).wait()
            pltpu.make_async_copy(
                const_hbm.at[h_v.at[b]], hexp_v.at[b], sems[b]).wait()

        def blend(c, b):
            for j in range(_LANES):
                h = hexp_v[b, j, pl.ds(0, nl)]
                for t in range(dim // nl):
                    lo = pair_v[b, j, pl.ds(t * nl, nl)]
                    hi = pair_v[b, j, pl.ds(dim + t * nl, nl)]
                    emb_v[b, j >> 1, pl.ds(dim * (j & 1) + t * nl, nl)] = (
                        lo + (hi - lo) * h)

        def put(c, b):
            pltpu.async_copy(
                emb_v.at[b],
                out2_hbm.at[pl.ds((row0 + c) * orpc, orpc)], wsems[b])

        def put_wait(c, b):
            pltpu.make_async_copy(
                emb_v.at[b],
                out2_hbm.at[pl.ds((row0 + c) * orpc, orpc)], wsems[b]).wait()

        prep(0, 0)
        fire(0, 0)
        prep(1, 1)
        fire(1, 1)

        def pair_step(p, _):
            c0 = p * 2
            for b in range(2):
                c = c0 + b

                @pl.when(p > 0)
                def _():
                    put_wait(c - 2, b)

                drain(c, b)
                blend(c, b)
                put(c, b)

                @pl.when(p + 1 < npair)
                def _():
                    prep(c + 2, b)
                    fire(c + 2, b)

            return 0

        lax.fori_loop(0, npair, pair_step, 0)
        put_wait(rows_per_w - 2, 0)
        put_wait(rows_per_w - 1, 1)

    return emb


def kernel(inputs, table):
    seq, batch = inputs.shape
    vocab, dim = table.shape
    n_idx = seq * batch
    flat_idx = inputs.reshape(n_idx // _LANES, _LANES)
    table2 = table.reshape(vocab * dim // 128, 128)
    const2 = jnp.tile(
        jnp.concatenate([jnp.zeros((1, 16), jnp.float32),
                         jnp.ones((1, 16), jnp.float32)]), (32, 1))
    out = _build(n_idx, vocab, dim)(flat_idx, table2, const2)
    return out.reshape(seq, batch, dim)


# speedup vs baseline: 4.3361x; 1.5593x over previous
import functools

import jax
import jax.numpy as jnp
from jax import lax
from jax.experimental import pallas as pl
from jax.experimental.pallas import tpu as pltpu
from jax.experimental.pallas import tpu_sc as plsc

_LANES = 128
_K = 5


@functools.lru_cache(maxsize=None)
def _build(n_idx, vocab, dim):
    info = plsc.get_sparse_core_info()
    nc, ns = info.num_cores, info.num_subcores
    nw = nc * ns
    rows_total = n_idx // _LANES
    rows_per_w = rows_total // nw
    nchunk = rows_per_w // _K
    npair = nchunk // 2

    mesh = plsc.VectorSubcoreMesh(core_axis_name="c", subcore_axis_name="s")

    @functools.partial(
        pl.kernel,
        mesh=mesh,
        out_type=jax.ShapeDtypeStruct((rows_total, _LANES, dim), jnp.float32),
        scratch_types=[
            pltpu.VMEM((rows_per_w, _LANES), jnp.int32),
            pltpu.VMEM((_K, _LANES, dim), jnp.float32),
            pltpu.VMEM((_K, _LANES, dim), jnp.float32),
            pltpu.SemaphoreType.DMA,
            pltpu.SemaphoreType.DMA,
        ],
        compiler_params=pltpu.CompilerParams(use_tc_tiling_on_sc=False),
    )
    def emb(idx_hbm, table_hbm, out_hbm, idx_v, rows0, rows1, sem0, sem1):
        wid = lax.axis_index("s") * nc + lax.axis_index("c")
        row0 = wid * rows_per_w
        pltpu.sync_copy(idx_hbm.at[pl.ds(row0, rows_per_w)], idx_v)

        def fire(chunk, buf, sem):
            for j in range(_K):
                pltpu.async_copy(
                    table_hbm.at[idx_v.at[chunk * _K + j]], buf.at[j], sem)

        def drain(chunk, buf, sem):
            for j in range(_K):
                pltpu.make_async_copy(
                    table_hbm.at[idx_v.at[chunk * _K + j]], buf.at[j],
                    sem).wait()

        def put(chunk, buf):
            pltpu.sync_copy(buf, out_hbm.at[pl.ds(row0 + chunk * _K, _K)])

        fire(0, rows0, sem0)

        def pair(p, _):
            c0 = p * 2
            drain(c0, rows0, sem0)
            fire(c0 + 1, rows1, sem1)
            put(c0, rows0)
            drain(c0 + 1, rows1, sem1)

            @pl.when(p + 1 < npair)
            def _():
                fire(c0 + 2, rows0, sem0)

            put(c0 + 1, rows1)
            return 0

        lax.fori_loop(0, npair, pair, 0)

    return emb


def kernel(inputs, table):
    seq, batch = inputs.shape
    vocab, dim = table.shape
    n_idx = seq * batch
    flat_idx = inputs.reshape(n_idx // _LANES, _LANES)
    out = _build(n_idx, vocab, dim)(flat_idx, table)
    return out.reshape(seq, batch, dim)
